# baseline (speedup 1.0000x reference)
import functools
import math

import jax
import jax.numpy as jnp
from jax import lax
from jax.experimental import pallas as pl
from jax.experimental.pallas import tpu as pltpu
from jax.experimental.pallas import tpu_sc as plsc

_LFREQ = 10000.0
_LANES = 16


def _sinusoidal_pe(length, dm):
    pos = jnp.arange(length, dtype=jnp.float32)[:, None]
    i = jnp.arange(0, dm, 2, dtype=jnp.float32)
    div = jnp.exp(-(jnp.log(_LFREQ)) * i / dm)
    angles = pos * div[None, :]
    pe = jnp.zeros((length, dm), dtype=jnp.float32)
    pe = pe.at[:, 0::2].set(jnp.sin(angles))
    pe = pe.at[:, 1::2].set(jnp.cos(angles))
    return pe


@functools.partial(jax.jit, static_argnames=("n_rows", "dm", "length"))
def _embed_sc(idx, pe, table, n_rows, dm, length):
    info = plsc.get_sparse_core_info()
    nc, ns = info.num_cores, info.num_subcores
    nw = nc * ns
    rows_per_w = n_rows // nw
    chunk = 64
    n_chunks = rows_per_w // chunk
    vecs_per_row = dm // _LANES
    scale = jnp.float32(math.sqrt(dm))

    mesh = plsc.VectorSubcoreMesh(core_axis_name="c", subcore_axis_name="s")

    @functools.partial(
        pl.kernel,
        out_type=jax.ShapeDtypeStruct((n_rows, dm), jnp.float32),
        mesh=mesh,
        scratch_types=[
            pltpu.VMEM((rows_per_w,), jnp.int32),
            pltpu.VMEM((length, dm), jnp.float32),
            pltpu.VMEM((chunk, dm), jnp.float32),
            pltpu.SemaphoreType.DMA,
        ],
    )
    def body(idx_hbm, pe_hbm, table_hbm, out_hbm, idx_v, pe_v, buf, sem):
        wid = lax.axis_index("s") * nc + lax.axis_index("c")
        base = wid * rows_per_w
        pltpu.sync_copy(pe_hbm, pe_v)
        pltpu.sync_copy(idx_hbm.at[pl.ds(base, rows_per_w)], idx_v)

        def chunk_body(c, _):
            s = base + c * chunk
            pltpu.async_copy(
                table_hbm.at[idx_v.at[pl.ds(c * chunk, chunk)]], buf, sem
            ).wait()

            def row_body(r, p):
                for j in range(vecs_per_row):
                    x = buf[r, pl.ds(j * _LANES, _LANES)]
                    pv = pe_v[p, pl.ds(j * _LANES, _LANES)]
                    buf[r, pl.ds(j * _LANES, _LANES)] = x * scale + pv
                p = p + 1
                return jnp.where(p >= length, 0, p)

            lax.fori_loop(0, chunk, row_body, lax.rem(s, length))
            pltpu.sync_copy(buf, out_hbm.at[pl.ds(s, chunk)])
            return 0

        lax.fori_loop(0, n_chunks, chunk_body, 0)

    return body(idx, pe, table)


def kernel(text, embed_table):
    b, l = text.shape
    v, dm = embed_table.shape
    idx = text.reshape(-1).astype(jnp.int32)
    pe = _sinusoidal_pe(l, dm)
    out = _embed_sc(idx, pe, embed_table, b * l, dm, l)
    return out.reshape(b, l, dm)


# speedup vs baseline: 1.1095x; 1.1095x over previous
import functools
import math

import jax
import jax.numpy as jnp
from jax import lax
from jax.experimental import pallas as pl
from jax.experimental.pallas import tpu as pltpu
from jax.experimental.pallas import tpu_sc as plsc

_LFREQ = 10000.0
_LANES = 16


def _sinusoidal_pe(length, dm):
    pos = jnp.arange(length, dtype=jnp.float32)[:, None]
    i = jnp.arange(0, dm, 2, dtype=jnp.float32)
    div = jnp.exp(-(jnp.log(_LFREQ)) * i / dm)
    angles = pos * div[None, :]
    pe = jnp.zeros((length, dm), dtype=jnp.float32)
    pe = pe.at[:, 0::2].set(jnp.sin(angles))
    pe = pe.at[:, 1::2].set(jnp.cos(angles))
    return pe


@functools.partial(jax.jit, static_argnames=("n_rows", "dm", "length"))
def _embed_sc(idx, pe, table, n_rows, dm, length):
    info = plsc.get_sparse_core_info()
    nc, ns = info.num_cores, info.num_subcores
    nw = nc * ns
    rows_per_w = n_rows // nw
    chunk = 80
    n_chunks = rows_per_w // chunk
    n_pairs = n_chunks // 2
    vecs_per_row = dm // _LANES
    scale = jnp.float32(math.sqrt(dm))

    mesh = plsc.VectorSubcoreMesh(core_axis_name="c", subcore_axis_name="s")

    @functools.partial(
        pl.kernel,
        out_type=jax.ShapeDtypeStruct((n_rows, dm), jnp.float32),
        mesh=mesh,
        scratch_types=[
            pltpu.VMEM((rows_per_w,), jnp.int32),
            pltpu.VMEM((length, dm), jnp.float32),
            pltpu.VMEM((chunk, dm), jnp.float32),
            pltpu.VMEM((chunk, dm), jnp.float32),
            pltpu.SemaphoreType.DMA,
            pltpu.SemaphoreType.DMA,
            pltpu.SemaphoreType.DMA,
            pltpu.SemaphoreType.DMA,
        ],
    )
    def body(idx_hbm, pe_hbm, table_hbm, out_hbm,
             idx_v, pe_v, buf0, buf1, gsem0, gsem1, osem0, osem1):
        wid = lax.axis_index("s") * nc + lax.axis_index("c")
        base = wid * rows_per_w
        pltpu.sync_copy(pe_hbm, pe_v)
        pltpu.sync_copy(idx_hbm.at[pl.ds(base, rows_per_w)], idx_v)

        def gather(c, buf, gsem):
            pltpu.make_async_copy(
                table_hbm.at[idx_v.at[pl.ds(c * chunk, chunk)]], buf, gsem
            ).start()

        def compute_and_flush(c, buf, gsem, osem):
            pltpu.make_async_copy(
                table_hbm.at[idx_v.at[pl.ds(c * chunk, chunk)]], buf, gsem
            ).wait()
            s = base + c * chunk

            def row_body(r, p):
                for j in range(vecs_per_row):
                    x = buf[r, pl.ds(j * _LANES, _LANES)]
                    pv = pe_v[p, pl.ds(j * _LANES, _LANES)]
                    buf[r, pl.ds(j * _LANES, _LANES)] = x * scale + pv
                p = p + 1
                return jnp.where(p >= length, 0, p)

            lax.fori_loop(0, chunk, row_body, lax.rem(s, length))
            pltpu.make_async_copy(buf, out_hbm.at[pl.ds(s, chunk)], osem).start()

        def drain_out(c, buf, osem):
            s = base + c * chunk
            pltpu.make_async_copy(buf, out_hbm.at[pl.ds(s, chunk)], osem).wait()

        gather(0, buf0, gsem0)
        gather(1, buf1, gsem1)

        def pair_body(k, _):
            c0 = 2 * k
            compute_and_flush(c0, buf0, gsem0, osem0)
            compute_and_flush(c0 + 1, buf1, gsem1, osem1)

            @pl.when(k + 1 < n_pairs)
            def _():
                drain_out(c0, buf0, osem0)
                gather(c0 + 2, buf0, gsem0)
                drain_out(c0 + 1, buf1, osem1)
                gather(c0 + 3, buf1, gsem1)

            return 0

        lax.fori_loop(0, n_pairs, pair_body, 0)
        drain_out(n_chunks - 2, buf0, osem0)
        drain_out(n_chunks - 1, buf1, osem1)

    return body(idx, pe, table)


def kernel(text, embed_table):
    b, l = text.shape
    v, dm = embed_table.shape
    idx = text.reshape(-1).astype(jnp.int32)
    pe = _sinusoidal_pe(l, dm)
    out = _embed_sc(idx, pe, embed_table, b * l, dm, l)
    return out.reshape(b, l, dm)


# speedup vs baseline: 1.1743x; 1.0584x over previous
import functools
import math

import jax
import jax.numpy as jnp
from jax import lax
from jax.experimental import pallas as pl
from jax.experimental.pallas import tpu as pltpu
from jax.experimental.pallas import tpu_sc as plsc

_LFREQ = 10000.0
_LANES = 16
_NBUF = 4
_CHUNK = 40


def _sinusoidal_pe(length, dm):
    pos = jnp.arange(length, dtype=jnp.float32)[:, None]
    i = jnp.arange(0, dm, 2, dtype=jnp.float32)
    div = jnp.exp(-(jnp.log(_LFREQ)) * i / dm)
    angles = pos * div[None, :]
    pe = jnp.zeros((length, dm), dtype=jnp.float32)
    pe = pe.at[:, 0::2].set(jnp.sin(angles))
    pe = pe.at[:, 1::2].set(jnp.cos(angles))
    return pe


@functools.partial(jax.jit, static_argnames=("n_rows", "dm", "length"))
def _embed_sc(idx, pe, table, n_rows, dm, length):
    info = plsc.get_sparse_core_info()
    nc, ns = info.num_cores, info.num_subcores
    nw = nc * ns
    rows_per_w = n_rows // nw
    n_chunks = rows_per_w // _CHUNK
    n_rounds = n_chunks // _NBUF
    vecs_per_row = dm // _LANES
    scale = jnp.float32(math.sqrt(dm))

    mesh = plsc.VectorSubcoreMesh(core_axis_name="c", subcore_axis_name="s")

    @functools.partial(
        pl.kernel,
        out_type=jax.ShapeDtypeStruct((n_rows, dm), jnp.float32),
        mesh=mesh,
        scratch_types=[
            pltpu.VMEM((rows_per_w,), jnp.int32),
            pltpu.VMEM((length, dm), jnp.float32),
        ]
        + [pltpu.VMEM((_CHUNK, dm), jnp.float32)] * _NBUF
        + [pltpu.SemaphoreType.DMA] * (2 * _NBUF),
    )
    def body(idx_hbm, pe_hbm, table_hbm, out_hbm, idx_v, pe_v, *bufs_sems):
        bufs = bufs_sems[:_NBUF]
        gsems = bufs_sems[_NBUF:2 * _NBUF]
        osems = bufs_sems[2 * _NBUF:]
        wid = lax.axis_index("s") * nc + lax.axis_index("c")
        base = wid * rows_per_w
        pltpu.sync_copy(pe_hbm, pe_v)
        pltpu.sync_copy(idx_hbm.at[pl.ds(base, rows_per_w)], idx_v)

        def gather(c, i):
            pltpu.make_async_copy(
                table_hbm.at[idx_v.at[pl.ds(c * _CHUNK, _CHUNK)]], bufs[i], gsems[i]
            ).start()

        def wait_gather(c, i):
            pltpu.make_async_copy(
                table_hbm.at[idx_v.at[pl.ds(c * _CHUNK, _CHUNK)]], bufs[i], gsems[i]
            ).wait()

        def out_copy(c, i):
            return pltpu.make_async_copy(
                bufs[i],
                out_hbm.at[pl.ds(base + c * _CHUNK, _CHUNK)],
                osems[i],
            )

        for i in range(_NBUF - 1):
            gather(i, i)

        def round_body(k, _):
            for i in range(_NBUF):
                c = _NBUF * k + i
                wait_gather(c, i)

                buf = bufs[i]

                def row_body(r, p, buf=buf):
                    for j in range(vecs_per_row):
                        x = buf[r, pl.ds(j * _LANES, _LANES)]
                        pv = pe_v[p, pl.ds(j * _LANES, _LANES)]
                        buf[r, pl.ds(j * _LANES, _LANES)] = x * scale + pv
                    p = p + 1
                    return jnp.where(p >= length, 0, p)

                lax.fori_loop(0, _CHUNK, row_body,
                              lax.rem(base + c * _CHUNK, length))
                out_copy(c, i).start()

                nb = (i + _NBUF - 1) % _NBUF

                @pl.when(c + _NBUF - 1 < n_chunks)
                def _(c=c, nb=nb):
                    @pl.when(c >= 1)
                    def _():
                        out_copy(c - 1, nb).wait()

                    gather(c + _NBUF - 1, nb)

            return 0

        lax.fori_loop(0, n_rounds, round_body, 0)
        for i in range(_NBUF):
            out_copy(n_chunks - _NBUF + i, i).wait()

    return body(idx, pe, table)


def kernel(text, embed_table):
    b, l = text.shape
    v, dm = embed_table.shape
    idx = text.reshape(-1).astype(jnp.int32)
    pe = _sinusoidal_pe(l, dm)
    out = _embed_sc(idx, pe, embed_table, b * l, dm, l)
    return out.reshape(b, l, dm)


# speedup vs baseline: 1.8511x; 1.5763x over previous
import functools
import math

import jax
import jax.numpy as jnp
from jax import lax
from jax.experimental import pallas as pl
from jax.experimental.pallas import tpu as pltpu
from jax.experimental.pallas import tpu_sc as plsc

_LFREQ = 10000.0
_LANES = 16
_CHUNK = 40


def _sinusoidal_pe(length, dm):
    pos = jnp.arange(length, dtype=jnp.float32)[:, None]
    i = jnp.arange(0, dm, 2, dtype=jnp.float32)
    div = jnp.exp(-(jnp.log(_LFREQ)) * i / dm)
    angles = pos * div[None, :]
    pe = jnp.zeros((length, dm), dtype=jnp.float32)
    pe = pe.at[:, 0::2].set(jnp.sin(angles))
    pe = pe.at[:, 1::2].set(jnp.cos(angles))
    return pe


@functools.partial(jax.jit, static_argnames=("n_rows", "dm", "length"))
def _embed_sc(idx, pe, table, n_rows, dm, length):
    info = plsc.get_sparse_core_info()
    nc, ns = info.num_cores, info.num_subcores
    nw = nc * ns
    rows_per_w = n_rows // nw
    n_chunks = rows_per_w // _CHUNK
    n_rounds = n_chunks // 2
    vecs_per_row = dm // _LANES
    scale = jnp.float32(math.sqrt(dm))

    mesh = plsc.VectorSubcoreMesh(core_axis_name="c", subcore_axis_name="s")

    @functools.partial(
        pl.kernel,
        out_type=jax.ShapeDtypeStruct((n_rows, dm), jnp.float32),
        mesh=mesh,
        scratch_types=[
            pltpu.VMEM((rows_per_w,), jnp.int32),
            pltpu.VMEM((length, dm), jnp.float32),
            pltpu.VMEM((_CHUNK, dm), jnp.float32),
            pltpu.VMEM((_CHUNK, dm), jnp.float32),
            pltpu.VMEM((_CHUNK, dm), jnp.float32),
            pltpu.VMEM((_CHUNK, dm), jnp.float32),
        ]
        + [pltpu.SemaphoreType.DMA] * 4,
    )
    def body(idx_hbm, pe_hbm, table_hbm, out_hbm, idx_v, pe_v,
             x0, x1, y0, y1, g0, g1, o0, o1):
        xs, ys, gs, os_ = (x0, x1), (y0, y1), (g0, g1), (o0, o1)
        wid = lax.axis_index("s") * nc + lax.axis_index("c")
        base = wid * rows_per_w
        pltpu.sync_copy(pe_hbm, pe_v)
        pltpu.sync_copy(idx_hbm.at[pl.ds(base, rows_per_w)], idx_v)

        def gather(c, i):
            return pltpu.make_async_copy(
                table_hbm.at[idx_v.at[pl.ds(c * _CHUNK, _CHUNK)]], xs[i], gs[i]
            )

        def out_copy(c, i):
            return pltpu.make_async_copy(
                ys[i], out_hbm.at[pl.ds(base + c * _CHUNK, _CHUNK)], os_[i]
            )

        gather(0, 0).start()

        def step(c, i):
            wait_g = gather(c, i)
            wait_g.wait()

            @pl.when(c + 1 < n_chunks)
            def _():
                gather(c + 1, 1 - i).start()

            @pl.when(c >= 2)
            def _():
                out_copy(c - 2, i).wait()

            x, y = xs[i], ys[i]

            def row_body(r, p):
                pes = [pe_v[p, pl.ds(j * _LANES, _LANES)]
                       for j in range(vecs_per_row)]
                for j in range(vecs_per_row):
                    sl = pl.ds(j * _LANES, _LANES)
                    y[r, sl] = x[r, sl] * scale + pes[j]
                p = p + 1
                return jnp.where(p >= length, 0, p)

            lax.fori_loop(0, _CHUNK, row_body,
                          lax.rem(base + c * _CHUNK, length))
            out_copy(c, i).start()

        def round_body(k, _):
            step(2 * k, 0)
            step(2 * k + 1, 1)
            return 0

        lax.fori_loop(0, n_rounds, round_body, 0)
        out_copy(n_chunks - 2, 0).wait()
        out_copy(n_chunks - 1, 1).wait()

    return body(idx, pe, table)


def kernel(text, embed_table):
    b, l = text.shape
    v, dm = embed_table.shape
    idx = text.reshape(-1).astype(jnp.int32)
    pe = _sinusoidal_pe(l, dm)
    out = _embed_sc(idx, pe, embed_table, b * l, dm, l)
    return out.reshape(b, l, dm)
